# baseline (speedup 1.0000x reference)
import jax
import jax.numpy as jnp
from jax import lax
from jax.experimental import pallas as pl
from jax.experimental.pallas import tpu as pltpu
from jax.experimental.pallas import tpu_sc as plsc

B, H, S, D = 2, 12, 2048, 768
TOPK = 6
NSUB = 16
L = 16
CHUNK = S // NSUB
NCH = CHUNK // L
BIGI = 2 ** 30
NEG = float("-inf")

_GD = lax.GatherDimensionNumbers(
    offset_dims=(), collapsed_slice_dims=(0,), start_index_map=(0,))


def _shuf(x, p):
    return lax.gather(x, p[:, None], _GD, (1,),
                      mode=lax.GatherScatterMode.PROMISE_IN_BOUNDS)


def _pick(av, ai, bv, bi):
    t = (av > bv) | ((av == bv) & (ai < bi))
    return jnp.where(t, av, bv), jnp.where(t, ai, bi)


def _top6_vec(vregs, iregs, iota):
    idt = iregs[0].dtype
    vals16 = jnp.full((L,), NEG, jnp.float32)
    idx16 = jnp.full((L,), BIGI, idt)
    v = list(vregs)
    n = len(v)
    for t in range(TOPK):
        m, mi = v[0], iregs[0]
        for i in range(1, n):
            m, mi = _pick(v[i], iregs[i], m, mi)
        for sh in (1, 2, 4, 8):
            p = iota ^ sh
            m, mi = _pick(m, mi, _shuf(m, p), _shuf(mi, p))
        vals16 = jnp.where(iota == t, m, vals16)
        idx16 = jnp.where(iota == t, mi, idx16)
        v = [jnp.where(iregs[i] == mi, NEG, v[i]) for i in range(n)]
    return vals16, idx16


def _scores_body(attn_ref, stage_ref, hbuf, st, sem):
    cid = lax.axis_index("c")
    sid = lax.axis_index("s")
    iota = lax.iota(jnp.int32, L)

    col0 = sid * CHUNK
    base = cid * (H * S * S)
    copies = [
        pltpu.async_copy(attn_ref.at[pl.ds(base + h * (S * S) + col0, CHUNK)],
                         hbuf.at[h], sem)
        for h in range(H)
    ]
    for c in copies:
        c.wait()

    v = []
    ir = []
    for i in range(NCH):
        acc = hbuf[0, pl.ds(i * L, L)]
        for h in range(1, H):
            acc = acc + hbuf[h, pl.ds(i * L, L)]
        gidx = col0 + i * L + iota
        acc = jnp.where(gidx == 0, NEG, acc)
        v.append(acc)
        ir.append(gidx)

    vals16, idx16 = _top6_vec(v, ir, iota)
    st[0] = vals16
    st[1] = idx16.astype(jnp.float32)
    pltpu.sync_copy(st, stage_ref.at[cid, sid])


def _select_body(stage_ref, tok_ref, out_ref, mv, idxv, rows, outb, sem):
    cid = lax.axis_index("c")
    sid = lax.axis_index("s")
    iota = lax.iota(jnp.int32, L)

    @pl.when(sid == 0)
    def _merge():
        pltpu.sync_copy(stage_ref.at[cid], mv)
        cv = [mv[j, 0] for j in range(NSUB)]
        ci = [mv[j, 1] for j in range(NSUB)]
        _, fi = _top6_vec(cv, ci, iota)
        fii = fi.astype(jnp.int32)
        idxr = jnp.where(iota < TOPK, fii + (cid * (S - 1) - 1), 0)
        idxr = jnp.minimum(jnp.maximum(idxr, 0), B * (S - 1) - 1)
        idxv[...] = idxr
        pltpu.async_copy(tok_ref.at[idxv], rows, sem).wait()
        for d in range(D // L):
            a = rows[0, pl.ds(d * L, L)]
            for t in range(1, TOPK):
                a = a + rows[t, pl.ds(d * L, L)]
            outb[pl.ds(d * L, L)] = a * (1.0 / TOPK)
        pltpu.sync_copy(outb, out_ref.at[cid])


_MESH = plsc.VectorSubcoreMesh(core_axis_name="c", subcore_axis_name="s")

_SCORES_FN = pl.kernel(
    _scores_body,
    out_type=jax.ShapeDtypeStruct((B, NSUB, 2, L), jnp.float32),
    mesh=_MESH,
    scratch_types=[
        pltpu.VMEM((H, CHUNK), jnp.float32),
        pltpu.VMEM((2, L), jnp.float32),
        pltpu.SemaphoreType.DMA,
    ],
)

_SELECT_FN = pl.kernel(
    _select_body,
    out_type=jax.ShapeDtypeStruct((B, D), jnp.float32),
    mesh=_MESH,
    scratch_types=[
        pltpu.VMEM((NSUB, 2, L), jnp.float32),
        pltpu.VMEM((L,), jnp.int32),
        pltpu.VMEM((L, D), jnp.float32),
        pltpu.VMEM((D,), jnp.float32),
        pltpu.SemaphoreType.DMA,
    ],
)


def kernel(attn_weights, tokens):
    attn_flat = attn_weights.reshape(-1)
    tok2 = tokens.reshape(B * (S - 1), D)
    staged = _SCORES_FN(attn_flat)
    return _SELECT_FN(staged, tok2)


# speedup vs baseline: 10.2617x; 10.2617x over previous
import jax
import jax.numpy as jnp
from jax import lax
from jax.experimental import pallas as pl
from jax.experimental.pallas import tpu as pltpu
from jax.experimental.pallas import tpu_sc as plsc

B, H, S, D = 2, 12, 2048, 768
TOPK = 6
NSUB = 16
L = 16
CHUNK = S // NSUB
NCH = CHUNK // L
BIGI = 2 ** 30
NEG = float("-inf")

_GD = lax.GatherDimensionNumbers(
    offset_dims=(), collapsed_slice_dims=(0,), start_index_map=(0,))


def _shuf(x, p):
    return lax.gather(x, p[:, None], _GD, (1,),
                      mode=lax.GatherScatterMode.PROMISE_IN_BOUNDS)


def _pick(av, ai, bv, bi):
    t = (av > bv) | ((av == bv) & (ai < bi))
    return jnp.where(t, av, bv), jnp.where(t, ai, bi)


def _top6_vec(vregs, iregs, iota):
    idt = iregs[0].dtype
    vals16 = jnp.full((L,), NEG, jnp.float32)
    idx16 = jnp.full((L,), BIGI, idt)
    v = list(vregs)
    n = len(v)
    for t in range(TOPK):
        m, mi = v[0], iregs[0]
        for i in range(1, n):
            m, mi = _pick(v[i], iregs[i], m, mi)
        for sh in (1, 2, 4, 8):
            p = iota ^ sh
            m, mi = _pick(m, mi, _shuf(m, p), _shuf(mi, p))
        vals16 = jnp.where(iota == t, m, vals16)
        idx16 = jnp.where(iota == t, mi, idx16)
        v = [jnp.where(iregs[i] == mi, NEG, v[i]) for i in range(n)]
    return vals16, idx16


def _scores_body(attn_ref, stage_ref, hbuf, st, sem):
    cid = lax.axis_index("c")
    sid = lax.axis_index("s")
    iota = lax.iota(jnp.int32, L)

    col0 = sid * CHUNK
    copies = [
        pltpu.async_copy(attn_ref.at[cid, h, 0, pl.ds(col0, CHUNK)],
                         hbuf.at[h], sem)
        for h in range(H)
    ]
    for c in copies:
        c.wait()

    v = []
    ir = []
    for i in range(NCH):
        acc = hbuf[0, pl.ds(i * L, L)]
        for h in range(1, H):
            acc = acc + hbuf[h, pl.ds(i * L, L)]
        gidx = col0 + i * L + iota
        acc = jnp.where(gidx == 0, NEG, acc)
        v.append(acc)
        ir.append(gidx)

    vals16, idx16 = _top6_vec(v, ir, iota)
    st[0] = vals16
    st[1] = idx16.astype(jnp.float32)
    pltpu.sync_copy(st, stage_ref.at[cid, sid])


def _select_body(stage_ref, tok_ref, out_ref, mv, idxv, rows, outb, sem):
    cid = lax.axis_index("c")
    sid = lax.axis_index("s")
    iota = lax.iota(jnp.int32, L)

    @pl.when(sid == 0)
    def _merge():
        pltpu.sync_copy(stage_ref.at[cid], mv)
        cv = [mv[j, 0] for j in range(NSUB)]
        ci = [mv[j, 1] for j in range(NSUB)]
        _, fi = _top6_vec(cv, ci, iota)
        fii = fi.astype(jnp.int32)
        idxr = jnp.where(iota < TOPK, fii - 1, 0)
        idxr = jnp.minimum(jnp.maximum(idxr, 0), S - 2)
        idxv[...] = idxr
        pltpu.async_copy(tok_ref.at[cid].at[idxv], rows, sem).wait()
        for d in range(D // L):
            a = rows[0, pl.ds(d * L, L)]
            for t in range(1, TOPK):
                a = a + rows[t, pl.ds(d * L, L)]
            outb[pl.ds(d * L, L)] = a * (1.0 / TOPK)
        pltpu.sync_copy(outb, out_ref.at[cid])


_MESH = plsc.VectorSubcoreMesh(core_axis_name="c", subcore_axis_name="s")

_SCORES_FN = pl.kernel(
    _scores_body,
    out_type=jax.ShapeDtypeStruct((B, NSUB, 2, L), jnp.float32),
    mesh=_MESH,
    scratch_types=[
        pltpu.VMEM((H, CHUNK), jnp.float32),
        pltpu.VMEM((2, L), jnp.float32),
        pltpu.SemaphoreType.DMA,
    ],
)

_SELECT_FN = pl.kernel(
    _select_body,
    out_type=jax.ShapeDtypeStruct((B, D), jnp.float32),
    mesh=_MESH,
    scratch_types=[
        pltpu.VMEM((NSUB, 2, L), jnp.float32),
        pltpu.VMEM((L,), jnp.int32),
        pltpu.VMEM((L, D), jnp.float32),
        pltpu.VMEM((D,), jnp.float32),
        pltpu.SemaphoreType.DMA,
    ],
)


def kernel(attn_weights, tokens):
    staged = _SCORES_FN(attn_weights)
    return _SELECT_FN(staged, tokens)
